# Initial kernel scaffold; baseline (speedup 1.0000x reference)
import jax
import jax.numpy as jnp
from jax.experimental import pallas as pl


def kernel(x, m, x_table, m_table, W, b, pe):
    raise NotImplementedError("write your pallas kernel here")


# speedup vs baseline: 2.7189x; 2.7189x over previous
import functools

import jax
import jax.numpy as jnp
from jax import lax
from jax.experimental import pallas as pl
from jax.experimental.pallas import tpu as pltpu
from jax.experimental.pallas import tpu_sc as plsc

_B = 1024
_L = 200
_D = 64
_N = _B * _L
_NC, _NS = 2, 16
_NW = _NC * _NS
_G = 128
_NGW = _N // (_NW * _G)

_SC_MESH = plsc.VectorSubcoreMesh(
    core_axis_name="c", subcore_axis_name="s", num_cores=_NC, num_subcores=_NS
)


@functools.partial(
    pl.kernel,
    out_type=jax.ShapeDtypeStruct((_N, _D), jnp.float32),
    mesh=_SC_MESH,
    scratch_types=[
        pltpu.VMEM((_NGW, _G), jnp.int32),
        pltpu.VMEM((_G, _D), jnp.float32),
        pltpu.SemaphoreType.DMA,
    ],
    compiler_params=pltpu.CompilerParams(use_tc_tiling_on_sc=False),
)
def _gather_scale(table, idx_hbm, out_hbm, idx_v, rows_v, sem):
    wid = lax.axis_index("s") * _NC + lax.axis_index("c")
    pltpu.sync_copy(idx_hbm.at[wid], idx_v)

    def grp(g, carry):
        pltpu.async_copy(table.at[idx_v.at[g]], rows_v, sem).wait()

        def tok(t, c2):
            for j in range(_D // 16):
                sl = pl.ds(j * 16, 16)
                rows_v[t, sl] = rows_v[t, sl] * 8.0
            return c2

        lax.fori_loop(0, _G, tok, 0, unroll=4)
        pltpu.sync_copy(rows_v, out_hbm.at[pl.ds((wid * _NGW + g) * _G, _G)])
        return carry

    lax.fori_loop(0, _NGW, grp, 0)


_KB = 32
_RB = _KB * (_L // 2)


def _dense_body(e_ref, pe_ref, w_ref, b_ref, out_ref):
    e = e_ref[...]
    xx = (e.reshape(_KB, _L // 2, 2 * _D) + pe_ref[...][None]).reshape(_RB, 2 * _D)
    r = jnp.dot(xx, w_ref[...], preferred_element_type=jnp.float32) + b_ref[...]
    out_ref[...] = xx * r


def _dense(emb2, pe2, w2, b2):
    grid = _B // _KB
    return pl.pallas_call(
        _dense_body,
        grid=(grid,),
        in_specs=[
            pl.BlockSpec((_RB, 2 * _D), lambda i: (i, 0)),
            pl.BlockSpec((_L // 2, 2 * _D), lambda i: (0, 0)),
            pl.BlockSpec((2 * _D, 2 * _D), lambda i: (0, 0)),
            pl.BlockSpec((1, 2 * _D), lambda i: (0, 0)),
        ],
        out_specs=pl.BlockSpec((_RB, 2 * _D), lambda i: (i, 0)),
        out_shape=jax.ShapeDtypeStruct((_N // 2, 2 * _D), jnp.float32),
    )(emb2, pe2, w2, b2)


def kernel(x, m, x_table, m_table, W, b, pe):
    x_idx = x.reshape(_NW, _NGW, _G).astype(jnp.int32)
    m_idx = m.reshape(_NW, _NGW, _G).astype(jnp.int32)

    emb_x = _gather_scale(x_table, x_idx)
    emb_m = _gather_scale(m_table, m_idx)

    pe2 = pe[0, :_L, :].reshape(_L // 2, 2 * _D)
    wt = W.T
    w2 = (
        jnp.zeros((2 * _D, 2 * _D), jnp.float32)
        .at[:_D, :_D].set(wt)
        .at[_D:, _D:].set(wt)
    )
    b2 = (jnp.concatenate([b, b]) + 1.0).reshape(1, 2 * _D)

    out2 = _dense(emb_x.reshape(_N // 2, 2 * _D), pe2, w2, b2)

    out = out2.reshape(_B, _L, _D)
    return (
        (out, emb_x.reshape(_B, _L, _D)),
        emb_m.reshape(_B, _L, _D),
    )


# speedup vs baseline: 4.8411x; 1.7806x over previous
import functools

import jax
import jax.numpy as jnp
from jax import lax
from jax.experimental import pallas as pl
from jax.experimental.pallas import tpu as pltpu
from jax.experimental.pallas import tpu_sc as plsc

_B = 1024
_L = 200
_D = 64
_V = 100000
_N = _B * _L
_NC, _NS = 2, 16
_NW = _NC * _NS
_G = 128
_GW = _N // (_NW * _G)
_SS = 5
_NSS = _GW // _SS


_VC = 12544


def _prep_body(x_ref, m_ref, out_ref):
    cat = jnp.concatenate([x_ref[...], m_ref[...]], axis=0)
    out_ref[...] = cat.T * 8.0


def _prep(xt_t, mt_t):
    return pl.pallas_call(
        _prep_body,
        grid=((_V + _VC - 1) // _VC,),
        in_specs=[
            pl.BlockSpec((_D, _VC), lambda j: (0, j)),
            pl.BlockSpec((_D, _VC), lambda j: (0, j)),
        ],
        out_specs=pl.BlockSpec((_VC, 2 * _D), lambda j: (j, 0)),
        out_shape=jax.ShapeDtypeStruct((_V, 2 * _D), jnp.float32),
    )(xt_t, mt_t)


_SC_MESH = plsc.VectorSubcoreMesh(
    core_axis_name="c", subcore_axis_name="s", num_cores=_NC, num_subcores=_NS
)


@functools.partial(
    pl.kernel,
    out_type=jax.ShapeDtypeStruct((_N, _D), jnp.float32),
    mesh=_SC_MESH,
    scratch_types=[
        pltpu.VMEM((_GW, _G), jnp.int32),
        pltpu.VMEM((_SS * _G, _D), jnp.float32),
        pltpu.VMEM((_SS * _G, _D), jnp.float32),
        pltpu.SemaphoreType.DMA,
        pltpu.SemaphoreType.DMA,
    ],
    compiler_params=pltpu.CompilerParams(use_tc_tiling_on_sc=False),
)
def _gather(table, idx_hbm, out_hbm, idx_v, buf_a, buf_b, sem_a, sem_b):
    c = lax.axis_index("c")
    s = lax.axis_index("s")
    w = c * _NS + s
    base = w * (_GW * _G)
    pltpu.sync_copy(idx_hbm.at[w], idx_v)

    bufs = (buf_a, buf_b)
    sems = (sem_a, sem_b)

    def fire(k):
        bb, ss = bufs[k % 2], sems[k % 2]
        return [
            pltpu.async_copy(
                table.at[idx_v.at[k * _SS + b]],
                bb.at[pl.ds(b * _G, _G)],
                ss,
            )
            for b in range(_SS)
        ]

    pending = fire(0)
    for k in range(_NSS):
        for h in pending:
            h.wait()
        pending = fire(k + 1) if k + 1 < _NSS else []
        pltpu.sync_copy(
            bufs[k % 2], out_hbm.at[pl.ds(base + k * _SS * _G, _SS * _G)]
        )


_BC = 128
_NJ = _B // _BC
_HR = 50
_RB = _HR * _BC


def _emit(o, ref):
    t = o.T.reshape(2, _D, _HR, _BC)
    ref[0] = jnp.transpose(t, (2, 0, 1, 3)).reshape(_RB, _BC)


def _dense_main_body(ex_ref, pe_ref, w_ref, b_ref, out_ref, ox_ref):
    ex = ex_ref[...]
    pe3 = pe_ref[0][:, None, :]
    xx = (ex.reshape(_HR, _BC, 2 * _D) + pe3).reshape(_RB, 2 * _D)
    r = jnp.dot(xx, w_ref[...], preferred_element_type=jnp.float32) + b_ref[...]
    _emit(xx * r, out_ref)
    _emit(ex, ox_ref)


def _dense_m_body(em_ref, om_ref):
    _emit(em_ref[...], om_ref)


_OUT_BLK = pl.BlockSpec((1, _RB, _BC), lambda j, hf: (hf, 0, j))
_IN_BLK = pl.BlockSpec((_RB, 2 * _D), lambda j, hf: (2 * j + hf, 0))


def _dense_main(gx2, pe2, w2, b2):
    out_sds = jax.ShapeDtypeStruct((2, _RB, _B), jnp.float32)
    return pl.pallas_call(
        _dense_main_body,
        grid=(_NJ, 2),
        in_specs=[
            _IN_BLK,
            pl.BlockSpec((1, _HR, 2 * _D), lambda j, hf: (hf, 0, 0)),
            pl.BlockSpec((2 * _D, 2 * _D), lambda j, hf: (0, 0)),
            pl.BlockSpec((1, 2 * _D), lambda j, hf: (0, 0)),
        ],
        out_specs=[_OUT_BLK, _OUT_BLK],
        out_shape=[out_sds, out_sds],
    )(gx2, pe2, w2, b2)


def _dense_m(gm2):
    out_sds = jax.ShapeDtypeStruct((2, _RB, _B), jnp.float32)
    return pl.pallas_call(
        _dense_m_body,
        grid=(_NJ, 2),
        in_specs=[_IN_BLK],
        out_specs=[_OUT_BLK],
        out_shape=[out_sds],
    )(gm2)


def _perm_tokens(a):
    return jnp.transpose(
        a.reshape(_NJ, _BC, _L // 2, 2), (0, 2, 1, 3)
    ).reshape(_NW, _GW, _G)


def kernel(x, m, x_table, m_table, W, b, pe):
    xq = _perm_tokens(x.astype(jnp.int32))
    mq = _perm_tokens(m.astype(jnp.int32))

    tables_pack = _prep(x_table.T, m_table.T)
    tflat = tables_pack.reshape(2 * _V, _D)
    gx = _gather(tflat, 2 * xq)
    gm = _gather(tflat, 2 * mq + 1)

    pe2 = pe[0, :_L, :].reshape(2, _L // 4, 2 * _D)
    wt = W.T
    w2 = (
        jnp.zeros((2 * _D, 2 * _D), jnp.float32)
        .at[:_D, :_D].set(wt)
        .at[_D:, _D:].set(wt)
    )
    b2 = (jnp.concatenate([b, b]) + 1.0).reshape(1, 2 * _D)

    out_p, ox_p = _dense_main(gx.reshape(_N // 2, 2 * _D), pe2, w2, b2)
    (om_p,) = _dense_m(gm.reshape(_N // 2, 2 * _D))

    def unpack(p):
        return jnp.transpose(p.reshape(_L, _D, _B), (2, 0, 1))

    return ((unpack(out_p), unpack(ox_p)), unpack(om_p))
